# baseline (speedup 1.0000x reference)
import functools

import jax
import jax.numpy as jnp
import numpy as np
from jax import lax
from jax.experimental import pallas as pl
from jax.experimental.pallas import tpu as pltpu
from jax.experimental.pallas import tpu_sc as plsc

_ANCHORS = np.array([[116.0, 90.0], [156.0, 198.0], [373.0, 326.0]], np.float32)

_NB, _NA, _NCH, _H, _W = 64, 3, 5, 128, 128
_PLANES = _NCH * _NB * _NA
_NWORKERS = 32
_PPW = _PLANES // _NWORKERS
_PLANE = _H * _W


def _sig16(v):
    e = jnp.exp(-v)
    d = 1.0 + e
    bits = lax.bitcast_convert_type(d, jnp.int32)
    y = lax.bitcast_convert_type(jnp.int32(0x7EF311C3) - bits, jnp.float32)
    y = y * (2.0 - d * y)
    y = y * (2.0 - d * y)
    return y


def _sc_body(x_hbm, out_hbm, bin0, bin1, bout0, bout1, sem_in, sem_out):
    wid = lax.axis_index("s") * 2 + lax.axis_index("c")
    iota = lax.iota(jnp.int32, 16)
    iotaf = iota.astype(jnp.float32)
    gxs = [iotaf + float(k * 16) for k in range(_W // 16)]
    bins = (bin0, bin1)
    bouts = (bout0, bout1)

    def plane_ids(i):
        p = wid * _PPW + i
        c = p // (_NB * _NA)
        r = p % (_NB * _NA)
        b = r // _NA
        a = r % _NA
        return c, b, a

    def in_copy(i, slot):
        c, b, a = plane_ids(i)
        r_in = b * (_NA * _NCH) + a * _NCH + c
        return pltpu.make_async_copy(
            x_hbm.at[pl.ds(r_in, 1), :, :],
            bins[slot],
            sem_in.at[slot],
        )

    def out_copy(i, slot):
        c, b, a = plane_ids(i)
        return pltpu.make_async_copy(
            bouts[slot],
            out_hbm.at[pl.ds(c, 1), pl.ds(b, 1), pl.ds(a * _PLANE, _PLANE)],
            sem_out.at[slot],
        )

    def compute(i, slot):
        c, b, a = plane_ids(i)
        bi = bins[slot]
        bo = bouts[slot]
        aw8 = jnp.where(a == 0, 116.0, jnp.where(a == 1, 156.0, 373.0))
        ah8 = jnp.where(a == 0, 90.0, jnp.where(a == 1, 198.0, 326.0))

        def row_sig_x(rq, carry):
            for dr in range(4):
                r = rq * 4 + dr
                for k in range(_W // 16):
                    v = bi[0, r, pl.ds(k * 16, 16)]
                    bo[0, 0, pl.ds(r * _W + k * 16, 16)] = (_sig16(v) + gxs[k]) * 8.0
            return carry

        def row_sig_y(rq, carry):
            for dr in range(4):
                r = rq * 4 + dr
                yf = r.astype(jnp.float32)
                for k in range(_W // 16):
                    v = bi[0, r, pl.ds(k * 16, 16)]
                    bo[0, 0, pl.ds(r * _W + k * 16, 16)] = (_sig16(v) + yf) * 8.0
            return carry

        def row_exp_w(rq, carry):
            for dr in range(4):
                r = rq * 4 + dr
                for k in range(_W // 16):
                    v = bi[0, r, pl.ds(k * 16, 16)]
                    bo[0, 0, pl.ds(r * _W + k * 16, 16)] = jnp.exp(v) * aw8
            return carry

        def row_exp_h(rq, carry):
            for dr in range(4):
                r = rq * 4 + dr
                for k in range(_W // 16):
                    v = bi[0, r, pl.ds(k * 16, 16)]
                    bo[0, 0, pl.ds(r * _W + k * 16, 16)] = jnp.exp(v) * ah8
            return carry

        def row_sig(rq, carry):
            for dr in range(4):
                r = rq * 4 + dr
                for k in range(_W // 16):
                    v = bi[0, r, pl.ds(k * 16, 16)]
                    bo[0, 0, pl.ds(r * _W + k * 16, 16)] = _sig16(v)
            return carry

        def loop(fn):
            return lambda: lax.fori_loop(0, _H // 4, fn, 0)

        lax.switch(
            c,
            [loop(row_sig_x), loop(row_sig_y), loop(row_exp_w),
             loop(row_exp_h), loop(row_sig)],
        )

    in_copy(0, 0).start()

    def pair(ph, carry):
        i0 = ph * 2
        i1 = ph * 2 + 1

        in_copy(i1, 1).start()
        in_copy(i0, 0).wait()

        @pl.when(i0 >= 2)
        def _():
            out_copy(i0 - 2, 0).wait()

        compute(i0, 0)
        out_copy(i0, 0).start()

        @pl.when(i0 + 2 < _PPW)
        def _():
            in_copy(i0 + 2, 0).start()

        in_copy(i1, 1).wait()

        @pl.when(i1 >= 2)
        def _():
            out_copy(i1 - 2, 1).wait()

        compute(i1, 1)
        out_copy(i1, 1).start()
        return carry

    lax.fori_loop(0, _PPW // 2, pair, 0)

    out_copy(_PPW - 2, 0).wait()
    out_copy(_PPW - 1, 1).wait()


@jax.jit
def kernel(input):
    mesh = plsc.VectorSubcoreMesh(core_axis_name="c", subcore_axis_name="s")
    f = functools.partial(
        pl.kernel,
        out_type=jax.ShapeDtypeStruct((_NCH, _NB, _NA * _PLANE), jnp.float32),
        mesh=mesh,
        scratch_types=[
            pltpu.VMEM((1, _H, _W), jnp.float32),
            pltpu.VMEM((1, _H, _W), jnp.float32),
            pltpu.VMEM((1, 1, _PLANE), jnp.float32),
            pltpu.VMEM((1, 1, _PLANE), jnp.float32),
            pltpu.SemaphoreType.DMA((2,)),
            pltpu.SemaphoreType.DMA((2,)),
        ],
        compiler_params=pltpu.CompilerParams(needs_layout_passes=False),
    )(_sc_body)
    out = f(input.reshape(_NB * _NA * _NCH, _H, _W))
    return jnp.transpose(out, (1, 2, 0))


# speedup vs baseline: 4.4545x; 4.4545x over previous
import functools

import jax
import jax.numpy as jnp
import numpy as np
from jax import lax
from jax.experimental import pallas as pl
from jax.experimental.pallas import tpu as pltpu
from jax.experimental.pallas import tpu_sc as plsc

_ANCHORS = np.array([[116.0, 90.0], [156.0, 198.0], [373.0, 326.0]], np.float32)

_NB, _NA, _NCH, _H, _W = 64, 3, 5, 128, 128
_PLANES = _NCH * _NB * _NA
_NWORKERS = 32
_PPW = _PLANES // _NWORKERS
_PLANE = _H * _W


def _sig16(v):
    e = jnp.exp(-v)
    d = 1.0 + e
    bits = lax.bitcast_convert_type(d, jnp.int32)
    y = lax.bitcast_convert_type(jnp.int32(0x7EF311C3) - bits, jnp.float32)
    y = y * (2.0 - d * y)
    y = y * (2.0 - d * y)
    return y


def _sc_body(x_hbm, out_hbm, bin0, bin1, bout0, bout1, sem_in, sem_out):
    wid = lax.axis_index("s") * 2 + lax.axis_index("c")
    iota = lax.iota(jnp.int32, 16)
    iotaf = iota.astype(jnp.float32)
    gxs = [iotaf + float(k * 16) for k in range(_W // 16)]
    bins = (bin0, bin1)
    bouts = (bout0, bout1)

    def plane_ids(i):
        p = wid * _PPW + i
        c = p // (_NB * _NA)
        r = p % (_NB * _NA)
        b = r // _NA
        a = r % _NA
        return c, b, a

    def in_copy(i, slot):
        c, b, a = plane_ids(i)
        r_in = b * (_NA * _NCH) + a * _NCH + c
        return pltpu.make_async_copy(
            x_hbm.at[pl.ds(r_in, 1), :, :],
            bins[slot],
            sem_in.at[slot],
        )

    def out_copy(i, slot):
        c, b, a = plane_ids(i)
        return pltpu.make_async_copy(
            bouts[slot],
            out_hbm.at[pl.ds(c, 1), pl.ds(b, 1), pl.ds(a * _PLANE, _PLANE)],
            sem_out.at[slot],
        )

    def compute(i, slot):
        c, b, a = plane_ids(i)
        bi = bins[slot]
        bo = bouts[slot]
        aw8 = jnp.where(a == 0, 116.0, jnp.where(a == 1, 156.0, 373.0))
        ah8 = jnp.where(a == 0, 90.0, jnp.where(a == 1, 198.0, 326.0))

        def row_sig_x(rq, carry):
            for dr in range(4):
                r = rq * 4 + dr
                for k in range(_W // 16):
                    v = bi[0, r, pl.ds(k * 16, 16)]
                    bo[0, 0, pl.ds(r * _W + k * 16, 16)] = (_sig16(v) + gxs[k]) * 8.0
            return carry

        def row_sig_y(rq, carry):
            for dr in range(4):
                r = rq * 4 + dr
                yf = r.astype(jnp.float32)
                for k in range(_W // 16):
                    v = bi[0, r, pl.ds(k * 16, 16)]
                    bo[0, 0, pl.ds(r * _W + k * 16, 16)] = (_sig16(v) + yf) * 8.0
            return carry

        def row_exp_w(rq, carry):
            for dr in range(4):
                r = rq * 4 + dr
                for k in range(_W // 16):
                    v = bi[0, r, pl.ds(k * 16, 16)]
                    bo[0, 0, pl.ds(r * _W + k * 16, 16)] = jnp.exp(v) * aw8
            return carry

        def row_exp_h(rq, carry):
            for dr in range(4):
                r = rq * 4 + dr
                for k in range(_W // 16):
                    v = bi[0, r, pl.ds(k * 16, 16)]
                    bo[0, 0, pl.ds(r * _W + k * 16, 16)] = jnp.exp(v) * ah8
            return carry

        def row_sig(rq, carry):
            for dr in range(4):
                r = rq * 4 + dr
                for k in range(_W // 16):
                    v = bi[0, r, pl.ds(k * 16, 16)]
                    bo[0, 0, pl.ds(r * _W + k * 16, 16)] = _sig16(v)
            return carry

        def loop(fn):
            def go():
                def body(rq):
                    fn(rq, 0)
                plsc.parallel_loop(0, _H // 4)(body)
            return go

        lax.switch(
            c,
            [loop(row_sig_x), loop(row_sig_y), loop(row_exp_w),
             loop(row_exp_h), loop(row_sig)],
        )

    in_copy(0, 0).start()

    def pair(ph, carry):
        i0 = ph * 2
        i1 = ph * 2 + 1

        in_copy(i1, 1).start()
        in_copy(i0, 0).wait()

        @pl.when(i0 >= 2)
        def _():
            out_copy(i0 - 2, 0).wait()

        compute(i0, 0)
        out_copy(i0, 0).start()

        @pl.when(i0 + 2 < _PPW)
        def _():
            in_copy(i0 + 2, 0).start()

        in_copy(i1, 1).wait()

        @pl.when(i1 >= 2)
        def _():
            out_copy(i1 - 2, 1).wait()

        compute(i1, 1)
        out_copy(i1, 1).start()
        return carry

    lax.fori_loop(0, _PPW // 2, pair, 0)

    out_copy(_PPW - 2, 0).wait()
    out_copy(_PPW - 1, 1).wait()


@jax.jit
def kernel(input):
    mesh = plsc.VectorSubcoreMesh(core_axis_name="c", subcore_axis_name="s")
    f = functools.partial(
        pl.kernel,
        out_type=jax.ShapeDtypeStruct((_NCH, _NB, _NA * _PLANE), jnp.float32),
        mesh=mesh,
        scratch_types=[
            pltpu.VMEM((1, _H, _W), jnp.float32),
            pltpu.VMEM((1, _H, _W), jnp.float32),
            pltpu.VMEM((1, 1, _PLANE), jnp.float32),
            pltpu.VMEM((1, 1, _PLANE), jnp.float32),
            pltpu.SemaphoreType.DMA((2,)),
            pltpu.SemaphoreType.DMA((2,)),
        ],
        compiler_params=pltpu.CompilerParams(needs_layout_passes=False),
    )(_sc_body)
    out = f(input.reshape(_NB * _NA * _NCH, _H, _W))
    return jnp.transpose(out, (1, 2, 0))
